# Initial kernel scaffold; baseline (speedup 1.0000x reference)
import jax
import jax.numpy as jnp
from jax.experimental import pallas as pl


def kernel(text_input, table, Wk_f, Wr_f, b_f, Wk_b, Wr_b, b_b, W1, b1, W2, b2):
    raise NotImplementedError("write your pallas kernel here")


# speedup vs baseline: 12.0275x; 12.0275x over previous
import functools

import jax
import jax.numpy as jnp
from jax import lax
from jax.experimental import pallas as pl
from jax.experimental.pallas import tpu as pltpu
from jax.experimental.pallas import tpu_sc as plsc


def _sc_gather(table, idx, chunk=320):
    n = idx.shape[0]
    d = table.shape[1]
    info = plsc.get_sparse_core_info()
    nw = info.num_cores * info.num_subcores
    per_w = n // nw
    n_chunks = per_w // chunk
    assert per_w % chunk == 0 and n % nw == 0 and n_chunks % 2 == 0
    mesh = plsc.VectorSubcoreMesh(core_axis_name="c", subcore_axis_name="s")

    @functools.partial(
        pl.kernel,
        mesh=mesh,
        out_type=jax.ShapeDtypeStruct((n, d), jnp.bfloat16),
        scratch_types=[
            pltpu.VMEM((per_w,), jnp.int32),
            pltpu.VMEM((chunk, d), jnp.bfloat16),
            pltpu.VMEM((chunk, d), jnp.bfloat16),
            pltpu.SemaphoreType.DMA,
            pltpu.SemaphoreType.DMA,
        ],
        compiler_params=pltpu.CompilerParams(use_tc_tiling_on_sc=False),
    )
    def k(table_hbm, idx_hbm, out_hbm, idx_v, rows0, rows1, g0, g1):
        wid = lax.axis_index("s") * info.num_cores + lax.axis_index("c")
        w_base = pl.multiple_of(wid * per_w, 8)
        pltpu.sync_copy(idx_hbm.at[pl.ds(w_base, per_w)], idx_v)

        def gather(i, buf, sem):
            off = pl.multiple_of(i * chunk, 8)
            return pltpu.make_async_copy(
                table_hbm.at[idx_v.at[pl.ds(off, chunk)]], buf, sem)

        gather(0, rows0, g0).start()

        def body(j, carry):
            i0 = j * 2
            gather(i0, rows0, g0).wait()
            gather(i0 + 1, rows1, g1).start()
            base0 = pl.multiple_of(w_base + i0 * chunk, 8)
            pltpu.sync_copy(rows0, out_hbm.at[pl.ds(base0, chunk)])
            gather(i0 + 1, rows1, g1).wait()

            @pl.when(i0 + 2 < n_chunks)
            def _():
                gather(i0 + 2, rows0, g0).start()

            base1 = pl.multiple_of(w_base + (i0 + 1) * chunk, 8)
            pltpu.sync_copy(rows1, out_hbm.at[pl.ds(base1, chunk)])
            return carry

        lax.fori_loop(0, n_chunks // 2, body, 0)

    return k(table, idx)


def _tc_detile(table_t, blk=16384):
    e_sz, v = table_t.shape

    def body(tt_ref, out_ref):
        out_ref[:, 0:e_sz] = tt_ref[...].T.astype(jnp.bfloat16)

    return pl.pallas_call(
        body,
        grid=(pl.cdiv(v, blk),),
        in_specs=[pl.BlockSpec((e_sz, blk), lambda i: (0, i))],
        out_specs=pl.BlockSpec((blk, 128), lambda i: (i, 0)),
        out_shape=jax.ShapeDtypeStruct((v, 128), jnp.bfloat16),
    )(table_t)


def _tc_bilstm(emb, wkf_t, wrf_t, bf2, wkb_t, wrb_t, bb2, w1_t, b12, w2_t, b22):
    t_len, b_sz, d_pad = emb.shape
    g_sz = wkf_t.shape[0]
    h_sz = g_sz // 4
    e_sz = wkf_t.shape[1]

    def body(xf_ref, xb_ref, wkf_ref, wrf_ref, bf_ref, wkb_ref, wrb_ref,
             bb_ref, w1_ref, b1_ref, w2_ref, b2_ref, out_ref,
             hf, cf, hb, cb):
        t = pl.program_id(0)

        @pl.when(t == 0)
        def _():
            hf[...] = jnp.zeros_like(hf)
            cf[...] = jnp.zeros_like(cf)
            hb[...] = jnp.zeros_like(hb)
            cb[...] = jnp.zeros_like(cb)

        def sig(x):
            return 0.5 * jnp.tanh(0.5 * x) + 0.5

        def step(x_ref, wk_ref, wr_ref, b_ref, h, c):
            xt = x_ref[0][:, 0:e_sz].T
            z = (jnp.dot(wk_ref[...], xt, preferred_element_type=jnp.float32)
                 + jnp.dot(wr_ref[...], h[...].astype(jnp.bfloat16),
                           preferred_element_type=jnp.float32)
                 + b_ref[...])
            i = sig(z[0:h_sz])
            f = sig(z[h_sz:2 * h_sz])
            g = jnp.tanh(z[2 * h_sz:3 * h_sz])
            o = sig(z[3 * h_sz:4 * h_sz])
            cn = f * c[...] + i * g
            c[...] = cn
            h[...] = o * jnp.tanh(cn)

        step(xf_ref, wkf_ref, wrf_ref, bf_ref, hf, cf)
        step(xb_ref, wkb_ref, wrb_ref, bb_ref, hb, cb)

        @pl.when(t == t_len - 1)
        def _():
            hcat = jnp.concatenate([hf[...], hb[...]], axis=0)
            x1 = jnp.dot(w1_ref[...], hcat,
                         preferred_element_type=jnp.float32) + b1_ref[...]
            x1 = jnp.where(x1 >= 0.0, x1, 0.2 * x1)
            x2 = jnp.dot(w2_ref[...], x1,
                         preferred_element_type=jnp.float32) + b2_ref[...]
            out_ref[...] = sig(x2)

    full = lambda a: pl.BlockSpec(a.shape, lambda t: (0,) * a.ndim)
    return pl.pallas_call(
        body,
        grid=(t_len,),
        in_specs=[
            pl.BlockSpec((1, b_sz, d_pad), lambda t: (t, 0, 0)),
            pl.BlockSpec((1, b_sz, d_pad), lambda t: (t_len - 1 - t, 0, 0)),
            full(wkf_t), full(wrf_t), full(bf2),
            full(wkb_t), full(wrb_t), full(bb2),
            full(w1_t), full(b12), full(w2_t), full(b22),
        ],
        out_specs=pl.BlockSpec((3, b_sz), lambda t: (0, 0)),
        out_shape=jax.ShapeDtypeStruct((3, b_sz), jnp.float32),
        scratch_shapes=[pltpu.VMEM((h_sz, b_sz), jnp.float32)] * 4,
    )(emb, emb, wkf_t, wrf_t, bf2, wkb_t, wrb_t, bb2, w1_t, b12, w2_t, b22)


def kernel(text_input, table, Wk_f, Wr_f, b_f, Wk_b, Wr_b, b_b, W1, b1, W2, b2):
    b_sz, t_len = text_input.shape
    e_sz = table.shape[1]

    idx = text_input.astype(jnp.int32).T.reshape(-1)
    table_rm = _tc_detile(table.T)
    emb = _sc_gather(table_rm, idx).reshape(t_len, b_sz, 128)

    bf16 = jnp.bfloat16
    out_t = _tc_bilstm(
        emb,
        Wk_f.T.astype(bf16), Wr_f.T.astype(bf16), b_f.reshape(-1, 1),
        Wk_b.T.astype(bf16), Wr_b.T.astype(bf16), b_b.reshape(-1, 1),
        W1.T, b1.reshape(-1, 1), W2.T, b2.reshape(-1, 1),
    )
    return out_t.T
